# baseline (speedup 1.0000x reference)
import functools

import jax
import jax.numpy as jnp
from jax import lax
from jax.experimental import pallas as pl
from jax.experimental.pallas import tpu as pltpu
from jax.experimental.pallas import tpu_sc as plsc

_N = 50000
_D = 128
_NUM_PATH = 3
_PATH_LEN = 4
_K = _NUM_PATH * _PATH_LEN
_NW = 32
_RPW = 1600
_N_PAD = _NW * _RPW
_C = 32
_GB = _C * _PATH_LEN
_NCH = _RPW // _C
_IPW = _RPW * _PATH_LEN

_mesh = plsc.VectorSubcoreMesh(core_axis_name="c", subcore_axis_name="s")


@functools.partial(
    pl.kernel,
    mesh=_mesh,
    out_type=jax.ShapeDtypeStruct((_N_PAD, _D), jnp.float32),
    scratch_types=[
        pltpu.VMEM((_NUM_PATH * _IPW,), jnp.int32),
        pltpu.VMEM((2, _NUM_PATH * _GB, _D), jnp.float32),
        pltpu.VMEM((2, _C, _D), jnp.float32),
        pltpu.VMEM((_PATH_LEN, _D), jnp.float32),
        pltpu.SemaphoreType.DMA,
        pltpu.SemaphoreType.DMA,
        pltpu.SemaphoreType.DMA,
        pltpu.SemaphoreType.DMA,
    ],
)
def _sc_gather_acc(feats_hbm, idx_hbm, pw_hbm, out_hbm,
                   idx_v, rows_v, out_v, pw_v, sg0, sg1, so0, so1):
    wid = lax.axis_index("s") * 2 + lax.axis_index("c")
    sgs = (sg0, sg1)
    sos = (so0, so1)
    pltpu.sync_copy(pw_hbm, pw_v)
    for i in range(_NUM_PATH):
        pltpu.sync_copy(
            idx_hbm.at[pl.ds(i * _N_PAD * _PATH_LEN + wid * _IPW, _IPW)],
            idx_v.at[pl.ds(i * _IPW, _IPW)])

    def gather_copies(ch, b):
        return [
            pltpu.make_async_copy(
                feats_hbm.at[idx_v.at[pl.ds(i * _IPW + ch * _GB, _GB)]],
                rows_v.at[b, pl.ds(i * _GB, _GB)],
                sgs[b])
            for i in range(_NUM_PATH)
        ]

    def out_copy(ch, b):
        row0 = wid * _RPW + ch * _C
        return pltpu.make_async_copy(
            out_v.at[b], out_hbm.at[pl.ds(row0, _C)], sos[b])

    def compute(ch, b):
        for v in range(_D // 16):
            sl = pl.ds(v * 16, 16)
            pws = tuple(pw_v[j, sl] for j in range(_PATH_LEN))

            def row_body(c, carry, _sl=sl, _pws=pws, _b=b):
                base = c * _PATH_LEN
                acc = rows_v[_b, base, _sl] * _pws[0]
                for j in range(1, _PATH_LEN):
                    acc = acc + rows_v[_b, base + j, _sl] * _pws[j]
                for i in range(1, _NUM_PATH):
                    for j in range(_PATH_LEN):
                        acc = acc + rows_v[_b, i * _GB + base + j, _sl] * _pws[j]
                out_v[_b, c, _sl] = acc
                return carry

            lax.fori_loop(0, _C, row_body, 0)

    def chunk_body(ch, carry):
        for cp in gather_copies(ch, 0):
            cp.start()
        for cp in gather_copies(ch, 0):
            cp.wait()
        compute(ch, 0)
        oc = out_copy(ch, 0)
        oc.start()
        oc.wait()
        return carry

    lax.fori_loop(0, _NCH, chunk_body, 0)


_BN = 2000


def _mm_body(x_ref, w_ref, o_ref):
    o_ref[...] = jnp.maximum(
        lax.dot_general(x_ref[...], w_ref[...],
                        (((1,), (1,)), ((), ())),
                        preferred_element_type=jnp.float32),
        0.0)


def _tc_mm_relu(x, w):
    return pl.pallas_call(
        _mm_body,
        grid=(_N // _BN,),
        in_specs=[
            pl.BlockSpec((_BN, _D), lambda i: (i, 0)),
            pl.BlockSpec((_D, _D), lambda i: (0, 0)),
        ],
        out_specs=pl.BlockSpec((_BN, _D), lambda i: (i, 0)),
        out_shape=jax.ShapeDtypeStruct((_N, _D), jnp.float32),
    )(x, w)


def kernel(feats, paths, init_feats, path_weight, fc_weight):
    del init_feats
    p32 = paths.astype(jnp.int32)
    p32 = jnp.pad(p32, ((0, 0), (0, _N_PAD - _N), (0, 0)))
    idx_flat = p32.reshape(-1)
    pw = path_weight[0] * (1.0 / _NUM_PATH)
    acc = _sc_gather_acc(feats, idx_flat, pw)
    return _tc_mm_relu(acc, fc_weight)


# speedup vs baseline: 1.2202x; 1.2202x over previous
import functools

import jax
import jax.numpy as jnp
from jax import lax
from jax.experimental import pallas as pl
from jax.experimental.pallas import tpu as pltpu
from jax.experimental.pallas import tpu_sc as plsc

_N = 50000
_D = 128
_NUM_PATH = 3
_PATH_LEN = 4
_K = _NUM_PATH * _PATH_LEN
_NW = 32
_RPW = 1600
_N_PAD = _NW * _RPW
_C = 32
_GB = _C * _PATH_LEN
_NCH = _RPW // _C

_mesh = plsc.VectorSubcoreMesh(core_axis_name="c", subcore_axis_name="s")


@functools.partial(
    pl.kernel,
    mesh=_mesh,
    out_type=jax.ShapeDtypeStruct((_N_PAD, _D), jnp.float32),
    scratch_types=[
        pltpu.VMEM((_NUM_PATH * _GB,), jnp.int32),
        pltpu.VMEM((_NUM_PATH * _GB,), jnp.int32),
        pltpu.VMEM((_NUM_PATH * _GB, _D), jnp.float32),
        pltpu.VMEM((_NUM_PATH * _GB, _D), jnp.float32),
        pltpu.VMEM((_C, _D), jnp.float32),
        pltpu.VMEM((_C, _D), jnp.float32),
        pltpu.VMEM((_PATH_LEN, _D), jnp.float32),
        pltpu.SemaphoreType.DMA,
        pltpu.SemaphoreType.DMA,
        pltpu.SemaphoreType.DMA,
        pltpu.SemaphoreType.DMA,
        pltpu.SemaphoreType.DMA,
        pltpu.SemaphoreType.DMA,
    ],
)
def _sc_gather_acc(feats_hbm, idx_hbm, pw_hbm, out_hbm,
                   idx_v0, idx_v1, rows_v0, rows_v1, out_v0, out_v1, pw_v,
                   si0, si1, sg0, sg1, so0, so1):
    wid = lax.axis_index("s") * 2 + lax.axis_index("c")
    idx_bufs = (idx_v0, idx_v1)
    rows_bufs = (rows_v0, rows_v1)
    out_bufs = (out_v0, out_v1)
    sis = (si0, si1)
    sgs = (sg0, sg1)
    sos = (so0, so1)
    pltpu.sync_copy(pw_hbm, pw_v)

    def idx_copies(ch, b):
        row0 = wid * _RPW + ch * _C
        return [
            pltpu.make_async_copy(
                idx_hbm.at[pl.ds(i * _N_PAD * _PATH_LEN + row0 * _PATH_LEN, _GB)],
                idx_bufs[b].at[pl.ds(i * _GB, _GB)],
                sis[b])
            for i in range(_NUM_PATH)
        ]

    def gather_copies(b):
        return [
            pltpu.make_async_copy(
                feats_hbm.at[idx_bufs[b].at[pl.ds(i * _GB, _GB)]],
                rows_bufs[b].at[pl.ds(i * _GB, _GB)],
                sgs[b])
            for i in range(_NUM_PATH)
        ]

    def out_copy(ch, b):
        row0 = wid * _RPW + ch * _C
        return pltpu.make_async_copy(
            out_bufs[b], out_hbm.at[pl.ds(row0, _C)], sos[b])

    def compute(b):
        rows_v = rows_bufs[b]
        out_v = out_bufs[b]
        for v in range(_D // 16):
            sl = pl.ds(v * 16, 16)
            pws = tuple(pw_v[j, sl] for j in range(_PATH_LEN))

            def row_body(c, carry, _sl=sl, _pws=pws, _rows=rows_v, _out=out_v):
                base = c * _PATH_LEN
                acc = _rows[base, _sl] * _pws[0]
                for j in range(1, _PATH_LEN):
                    acc = acc + _rows[base + j, _sl] * _pws[j]
                for i in range(1, _NUM_PATH):
                    for j in range(_PATH_LEN):
                        acc = acc + _rows[i * _GB + base + j, _sl] * _pws[j]
                _out[c, _sl] = acc
                return carry

            lax.fori_loop(0, _C, row_body, 0)

    for cp in idx_copies(0, 0):
        cp.start()
    for cp in idx_copies(1, 1):
        cp.start()
    for cp in idx_copies(0, 0):
        cp.wait()
    for cp in gather_copies(0):
        cp.start()

    def pair_body(p, carry):
        for b in range(2):
            ch = p * 2 + b
            nb = 1 - b
            if b == 0:
                for cp in idx_copies(ch + 1, nb):
                    cp.wait()
                for cp in gather_copies(nb):
                    cp.start()
            else:
                @pl.when(ch + 1 < _NCH)
                def _():
                    for cp in idx_copies(ch + 1, nb):
                        cp.wait()
                    for cp in gather_copies(nb):
                        cp.start()
            for cp in gather_copies(b):
                cp.wait()

            @pl.when(ch + 2 < _NCH)
            def _():
                for cp in idx_copies(ch + 2, b):
                    cp.start()

            @pl.when(p >= 1)
            def _():
                out_copy(ch - 2, b).wait()

            compute(b)
            out_copy(ch, b).start()
        return carry

    lax.fori_loop(0, _NCH // 2, pair_body, 0)
    out_copy(_NCH - 2, 0).wait()
    out_copy(_NCH - 1, 1).wait()


_BN = 2000


def _mm_body(x_ref, w_ref, o_ref):
    o_ref[...] = jnp.maximum(
        lax.dot_general(x_ref[...], w_ref[...],
                        (((1,), (1,)), ((), ())),
                        preferred_element_type=jnp.float32),
        0.0)


def _tc_mm_relu(x, w):
    return pl.pallas_call(
        _mm_body,
        grid=(_N // _BN,),
        in_specs=[
            pl.BlockSpec((_BN, _D), lambda i: (i, 0)),
            pl.BlockSpec((_D, _D), lambda i: (0, 0)),
        ],
        out_specs=pl.BlockSpec((_BN, _D), lambda i: (i, 0)),
        out_shape=jax.ShapeDtypeStruct((_N, _D), jnp.float32),
    )(x, w)


def kernel(feats, paths, init_feats, path_weight, fc_weight):
    del init_feats
    p32 = paths.astype(jnp.int32)
    p32 = jnp.pad(p32, ((0, 0), (0, _N_PAD - _N), (0, 0)))
    idx_flat = p32.reshape(-1)
    pw = path_weight[0] * (1.0 / _NUM_PATH)
    acc = _sc_gather_acc(feats, idx_flat, pw)
    return _tc_mm_relu(acc, fc_weight)


# speedup vs baseline: 1.9591x; 1.6056x over previous
import functools

import jax
import jax.numpy as jnp
from jax import lax
from jax.experimental import pallas as pl
from jax.experimental.pallas import tpu as pltpu
from jax.experimental.pallas import tpu_sc as plsc

_N = 50000
_D = 128
_NUM_PATH = 3
_PATH_LEN = 4
_K = _NUM_PATH * _PATH_LEN
_NW = 32
_RPW = 1568
_N_PAD = _NW * _RPW
_C = 32
_NCH = _RPW // _C
_IDXR_CHUNK = _C * _K // 128

_mesh = plsc.VectorSubcoreMesh(core_axis_name="c", subcore_axis_name="s")


@functools.partial(
    pl.kernel,
    mesh=_mesh,
    out_type=jax.ShapeDtypeStruct((_N_PAD, _D), jnp.float32),
    scratch_types=[
        pltpu.VMEM((_C * _K,), jnp.int32),
        pltpu.VMEM((_C * _K, _D), jnp.float32),
        pltpu.VMEM((_C, _D), jnp.float32),
        pltpu.VMEM((_PATH_LEN, _D), jnp.float32),
        pltpu.SemaphoreType.DMA,
    ],
)
def _sc_gather_acc(feats_hbm, idx_hbm, pw_hbm, out_hbm,
                   idx_v, rows_v, out_v, pw_v, sem):
    wid = lax.axis_index("s") * 2 + lax.axis_index("c")
    pltpu.sync_copy(pw_hbm, pw_v)

    def chunk_body(ch, carry):
        row0 = wid * _RPW + ch * _C
        pltpu.sync_copy(idx_hbm.at[pl.ds(row0 * _K, _C * _K)], idx_v)
        copies = []
        for g in range(_IDXR_CHUNK):
            copies.append(pltpu.async_copy(
                feats_hbm.at[idx_v.at[pl.ds(g * 128, 128)]],
                rows_v.at[pl.ds(g * 128, 128)],
                sem))
        for cp in copies:
            cp.wait()
        for v in range(_D // 16):
            sl = pl.ds(v * 16, 16)
            pws = tuple(pw_v[j, sl] for j in range(_PATH_LEN))

            def row_body(c, acc_carry, _sl=sl, _pws=pws):
                b = c * _K
                acc = rows_v[b, _sl] * _pws[0]
                for k in range(1, _K):
                    acc = acc + rows_v[b + k, _sl] * _pws[k % _PATH_LEN]
                out_v[c, _sl] = acc
                return acc_carry

            lax.fori_loop(0, _C, row_body, 0)
        pltpu.sync_copy(out_v, out_hbm.at[pl.ds(row0, _C)])
        return carry

    lax.fori_loop(0, _NCH, chunk_body, 0)


_BN = 2000


def _mm_body(x_ref, w_ref, o_ref):
    o_ref[...] = jnp.maximum(
        lax.dot_general(x_ref[...], w_ref[...],
                        (((1,), (1,)), ((), ())),
                        preferred_element_type=jnp.float32),
        0.0)


def _tc_mm_relu(x, w):
    return pl.pallas_call(
        _mm_body,
        grid=(_N // _BN,),
        in_specs=[
            pl.BlockSpec((_BN, _D), lambda i: (i, 0)),
            pl.BlockSpec((_D, _D), lambda i: (0, 0)),
        ],
        out_specs=pl.BlockSpec((_BN, _D), lambda i: (i, 0)),
        out_shape=jax.ShapeDtypeStruct((_N, _D), jnp.float32),
    )(x, w)


def kernel(feats, paths, init_feats, path_weight, fc_weight):
    del init_feats
    idx = jnp.transpose(paths, (1, 0, 2)).reshape(_N, _K).astype(jnp.int32)
    idx = jnp.pad(idx, ((0, _N_PAD - _N), (0, 0)))
    idx_flat = idx.reshape(-1)
    pw = path_weight[0] * (1.0 / _NUM_PATH)
    acc = _sc_gather_acc(feats, idx_flat, pw)
    return _tc_mm_relu(acc, fc_weight)
